# baseline (speedup 1.0000x reference)
import functools

import jax
import jax.numpy as jnp
from jax import lax
from jax.experimental import pallas as pl
from jax.experimental.pallas import tpu as pltpu

D = 1024
RT = 128


def _recon_body(idx_ref, hi_ref, his_ref, cc_ref, ss_ref, out_ref):
    g = pl.program_id(0)

    def row(r, _):
        p = idx_ref[g * RT + r]
        hi = p >> 6
        lo = p & 63
        out_ref[pl.ds(r, 1)] = (
            hi_ref[pl.ds(hi, 1)] * cc_ref[pl.ds(lo, 1)]
            + his_ref[pl.ds(hi, 1)] * ss_ref[pl.ds(lo, 1)]
        )
        return 0

    lax.fori_loop(0, RT, row, 0, unroll=8)


def _make_recon(n_idx):
    tab_spec = pl.BlockSpec((128, 8, 128), lambda i, idx_ref: (0, 0, 0))
    lo_spec = pl.BlockSpec((64, 8, 128), lambda i, idx_ref: (0, 0, 0))
    return pl.pallas_call(
        _recon_body,
        grid_spec=pltpu.PrefetchScalarGridSpec(
            num_scalar_prefetch=1,
            grid=(n_idx // RT,),
            in_specs=[tab_spec, tab_spec, lo_spec, lo_spec],
            out_specs=pl.BlockSpec((RT, 8, 128), lambda i, idx_ref: (i, 0, 0)),
        ),
        out_shape=jax.ShapeDtypeStruct((n_idx, 8, 128), jnp.float32),
    )


def kernel(position_ids, table):
    pos = position_ids.reshape(-1)
    n = pos.shape[0]
    jj = jnp.arange(D)
    lo_t = table[:64]
    hi_t = table[::64]
    his = hi_t[:, jj ^ 1]
    cc = lo_t[:, jj | 1]
    sign = jnp.where(jj % 2 == 0, 1.0, -1.0).astype(jnp.float32)
    ss = lo_t[:, jj & ~1] * sign

    out = _make_recon(n)(
        pos,
        hi_t.reshape(-1, 8, 128),
        his.reshape(-1, 8, 128),
        cc.reshape(-1, 8, 128),
        ss.reshape(-1, 8, 128),
    )
    return out.reshape(position_ids.shape + (D,))


# speedup vs baseline: 2.9366x; 2.9366x over previous
import functools

import jax
import jax.numpy as jnp
from jax import lax
from jax.experimental import pallas as pl
from jax.experimental.pallas import tpu as pltpu
from jax.experimental.pallas import tpu_sc as plsc

D = 1024
NC, NS = 2, 16
NW = NC * NS
CHUNK = 8


def _make_gather(n_idx):
    b_per_w = n_idx // NW
    n_chunks = b_per_w // CHUNK
    n_iters = n_chunks // 4
    mesh = plsc.VectorSubcoreMesh(core_axis_name="c", subcore_axis_name="s")

    @functools.partial(
        pl.kernel,
        mesh=mesh,
        out_type=jax.ShapeDtypeStruct((n_idx, D), jnp.float32),
        scratch_types=[
            pltpu.VMEM((b_per_w,), jnp.int32),
            pltpu.VMEM((CHUNK, D), jnp.float32),
            pltpu.VMEM((CHUNK, D), jnp.float32),
            pltpu.VMEM((CHUNK, D), jnp.float32),
            pltpu.VMEM((CHUNK, D), jnp.float32),
            pltpu.SemaphoreType.DMA,
            pltpu.SemaphoreType.DMA,
            pltpu.SemaphoreType.DMA,
            pltpu.SemaphoreType.DMA,
            pltpu.SemaphoreType.DMA,
            pltpu.SemaphoreType.DMA,
            pltpu.SemaphoreType.DMA,
            pltpu.SemaphoreType.DMA,
        ],
    )
    def gather(pos_hbm, table_hbm, out_hbm, idx_v,
               r0, r1, r2, r3, gs0, gs1, gs2, gs3, ws0, ws1, ws2, ws3):
        wid = lax.axis_index("s") * NC + lax.axis_index("c")
        base = wid * b_per_w
        pltpu.sync_copy(pos_hbm.at[pl.ds(base, b_per_w)], idx_v)

        def g_src(i):
            return table_hbm.at[idx_v.at[pl.ds(i * CHUNK, CHUNK)]]

        def w_dst(i):
            return out_hbm.at[pl.ds(base + i * CHUNK, CHUNK)]

        pltpu.async_copy(g_src(0), r0, gs0)
        pltpu.async_copy(g_src(1), r1, gs1)

        def step(p, _):
            i = 4 * p

            @pl.when(p > 0)
            def _():
                pltpu.make_async_copy(r2, w_dst(i - 2), ws2).wait()
                pltpu.make_async_copy(r3, w_dst(i - 1), ws3).wait()

            pltpu.async_copy(g_src(i + 2), r2, gs2)
            pltpu.async_copy(g_src(i + 3), r3, gs3)

            pltpu.make_async_copy(g_src(i), r0, gs0).wait()
            pltpu.async_copy(r0, w_dst(i), ws0)
            pltpu.make_async_copy(g_src(i + 1), r1, gs1).wait()
            pltpu.async_copy(r1, w_dst(i + 1), ws1)

            pltpu.make_async_copy(r0, w_dst(i), ws0).wait()
            pltpu.make_async_copy(r1, w_dst(i + 1), ws1).wait()

            @pl.when(p + 1 < n_iters)
            def _():
                pltpu.async_copy(g_src(i + 4), r0, gs0)
                pltpu.async_copy(g_src(i + 5), r1, gs1)

            pltpu.make_async_copy(g_src(i + 2), r2, gs2).wait()
            pltpu.async_copy(r2, w_dst(i + 2), ws2)
            pltpu.make_async_copy(g_src(i + 3), r3, gs3).wait()
            pltpu.async_copy(r3, w_dst(i + 3), ws3)

            return 0

        lax.fori_loop(0, n_iters, step, 0)

        last = n_chunks - 2
        pltpu.make_async_copy(r2, w_dst(last), ws2).wait()
        pltpu.make_async_copy(r3, w_dst(last + 1), ws3).wait()

    return gather


def kernel(position_ids, table):
    pos = position_ids.reshape(-1)
    out = _make_gather(pos.shape[0])(pos, table)
    return out.reshape(position_ids.shape + (table.shape[1],))
